# baseline (speedup 1.0000x reference)
import functools

import jax
import jax.numpy as jnp
from jax import lax
from jax.experimental import pallas as pl
from jax.experimental.pallas import tpu as pltpu
from jax.experimental.pallas import tpu_sc as plsc

_B, _N, _P, _C = 16, 4096, 1024, 128
_S = 33
_OUTC = _C + 2
_R2 = 0.2 * 0.2
_NSLOTS = _P * _S
_HSLOTS = 512 * _S
_CH_SPLIT = 52


def _rsqrt(x, iters=3):
    i = lax.bitcast_convert_type(x, jnp.int32)
    i = jnp.int32(0x5F3759DF) - lax.shift_right_logical(i, 1)
    y = lax.bitcast_convert_type(i, jnp.float32)
    for _ in range(iters):
        y = y * (jnp.float32(1.5) - jnp.float32(0.5) * x * y * y)
    return y


def _acos(x):
    ax = jnp.abs(x)
    w = jnp.float32(1.0) - ax
    sw = w * _rsqrt(w)
    p = jnp.float32(-0.0012624911)
    for c in (0.0066700901, -0.0170881256, 0.0308918810, -0.0501743046,
              0.0889789874, -0.2145988016, 1.5707963050):
        p = p * ax + jnp.float32(c)
    t = sw * p
    return jnp.where(x >= jnp.float32(0.0), t, jnp.float32(3.1415927410125732) - t)


@functools.cache
def _build_kernel():
    mesh = plsc.VectorSubcoreMesh(core_axis_name="c", subcore_axis_name="s")

    @functools.partial(
        pl.kernel,
        out_type=(jax.ShapeDtypeStruct((_OUTC * _S, 2, 8, 8, 128), jnp.float32),
                  jax.ShapeDtypeStruct((_B * _P * _S,), jnp.int32)),
        mesh=mesh,
        compiler_params=pltpu.CompilerParams(needs_layout_passes=False),
        scratch_types=[
            pltpu.VMEM((3 * _N,), jnp.float32),
            pltpu.VMEM((3 * 512 + 16,), jnp.float32),
            pltpu.VMEM((512,), jnp.int32),
            pltpu.VMEM((_NSLOTS + 544,), jnp.int32),
            pltpu.VMEM((_S, 8, 128), jnp.float32),
            pltpu.VMEM((_S, 8, 128), jnp.float32),
            pltpu.VMEM((32, 128), jnp.float32),
            pltpu.VMEM((32, 128), jnp.float32),
            pltpu.SemaphoreType.DMA,
            pltpu.SemaphoreType.DMA,
            pltpu.SemaphoreType.DMA,
            pltpu.SemaphoreType.DMA,
        ],
    )
    def fused(xyz_hbm, cent_hbm, fps_hbm, feat_hbm, out_hbm, idx_hbm,
              xyzb, cbuf, fpsb, idxb, buf0, buf1, tab0, tab1,
              sem_in0, sem_in1, sem_out0, sem_out1):
        cid = lax.axis_index("c")
        sid = lax.axis_index("s")
        wid = cid * 16 + sid
        b = wid // 2
        h = wid % 2
        bt = b // 8
        bi = b % 8
        iota = lax.iota(jnp.int32, 16)
        iota33 = iota * _S
        ones16 = jnp.full((16,), 1, jnp.int32)
        r2 = jnp.float32(_R2)

        pltpu.sync_copy(xyz_hbm.at[pl.ds(b * (3 * _N), 3 * _N)], xyzb)
        for k in range(3):
            pltpu.sync_copy(
                cent_hbm.at[pl.ds(b * (3 * _P) + k * _P + h * 512, 512)],
                cbuf.at[pl.ds(k * 512, 512)])
        pltpu.sync_copy(fps_hbm.at[pl.ds(b * _P + h * 512, 512)], fpsb)

        def per_query(q, carry):
            cx = cbuf[pl.ds(q, 16)][0]
            cy = cbuf[pl.ds(512 + q, 16)][0]
            cz = cbuf[pl.ds(1024 + q, 16)][0]
            qv0 = h * _HSLOTS + q * _S
            lim = qv0 + 32

            def scan(lo, hi, cnt):
                @plsc.parallel_loop(lo, hi, unroll=16, carry=cnt)
                def supergroup(i, cnt):
                    base = i * 16
                    xv = xyzb[pl.ds(base, 16)]
                    yv = xyzb[pl.ds(_N + base, 16)]
                    zv = xyzb[pl.ds(2 * _N + base, 16)]
                    dx = xv - cx
                    dy = yv - cy
                    dz = zv - cz
                    d2 = dx * dx + dy * dy + dz * dz
                    m = d2 < r2
                    plsc.store_scatter(idxb,
                                       [cnt + plsc.cumsum(ones16, mask=m)],
                                       iota + base, mask=m)
                    return cnt + plsc.all_reduce_population_count(m)

                return supergroup

            def cond(st):
                g, cnt = st
                return jnp.logical_and(cnt[0] < lim, g < _N // 512)

            def body(st):
                g, cnt = st
                return g + jnp.int32(1), scan(g * 32, (g + 1) * 32, cnt)

            cnt = scan(0, 64, jnp.broadcast_to(qv0, (16,)))
            _, cnt = lax.while_loop(cond, body, (jnp.int32(2), cnt))
            cnt0 = cnt[0] - qv0
            first = jnp.where(cnt0 > 0,
                              idxb[pl.ds(qv0 + 1, 16)][0],
                              jnp.int32(0))
            firstv = jnp.broadcast_to(first, (16,))
            for u in range(2):
                lanes = iota + (u * 16)
                plsc.store_scatter(idxb, [qv0 + 1 + lanes], firstv,
                                   mask=lanes >= cnt0)
            return carry

        lax.fori_loop(0, 512, per_query, jnp.int32(0))

        @plsc.parallel_loop(0, 32)
        def _fps(i):
            fv = fpsb[pl.ds(i * 16, 16)]
            plsc.store_scatter(idxb, [h * _HSLOTS + (iota + i * 16) * _S], fv)

        @plsc.parallel_loop(0, 32)
        def _norm_centers(j):
            cx = cbuf[pl.ds(j * 16, 16)]
            cy = cbuf[pl.ds(512 + j * 16, 16)]
            cz = cbuf[pl.ds(1024 + j * 16, 16)]
            c2 = cx * cx + cy * cy + cz * cz
            n = c2 * _rsqrt(c2)
            r = _rsqrt(n + jnp.float32(1e-6))
            inv = r * r
            cbuf[pl.ds(j * 16, 16)] = cx * inv
            cbuf[pl.ds(512 + j * 16, 16)] = cy * inv
            cbuf[pl.ds(1024 + j * 16, 16)] = cz * inv

        @pl.loop(0, _S)
        def _inv_s(s):
            sbase = (h * 512) * _S + s

            @plsc.parallel_loop(0, 32, unroll=4)
            def _inv_chunk(r):
                pv = iota + r * 16
                n = plsc.load_gather(idxb, [iota33 + (r * 16 * _S + sbase)])
                gx = plsc.load_gather(xyzb, [n])
                gy = plsc.load_gather(xyzb, [n + _N])
                gz = plsc.load_gather(xyzb, [n + 2 * _N])
                cnx = plsc.load_gather(cbuf, [pv])
                cny = plsc.load_gather(cbuf, [pv + 512])
                cnz = plsc.load_gather(cbuf, [pv + 1024])
                s2 = gx * gx + gy * gy + gz * gz + jnp.float32(1e-12)
                rr = _rsqrt(s2)
                cross = (cnx * gx + cny * gy + cnz * gz) * rr
                cross = jnp.minimum(
                    jnp.maximum(cross, jnp.float32(-1 + 1e-6)),
                    jnp.float32(1 - 1e-6))
                pt = h * 4 + r // 8
                po = (r % 8) * 16
                buf0[s, pt, pl.ds(po, 16)] = s2 * rr
                buf1[s, pt, pl.ds(po, 16)] = _acos(cross)

        pltpu.sync_copy(buf0.at[:, pl.ds(h * 4, 4), :],
                        out_hbm.at[pl.ds(0, _S), bt, pl.ds(h * 4, 4), bi, :])
        pltpu.sync_copy(buf1.at[:, pl.ds(h * 4, 4), :],
                        out_hbm.at[pl.ds(_S, _S), bt, pl.ds(h * 4, 4), bi, :])

        base = (b * _P) * _S
        pltpu.sync_copy(idxb.at[pl.ds(h * _HSLOTS, _HSLOTS)],
                        idx_hbm.at[pl.ds(base + h * _HSLOTS, _HSLOTS)])
        plsc.subcore_barrier()
        pltpu.sync_copy(idx_hbm.at[pl.ds(base + (1 - h) * _HSLOTS, _HSLOTS)],
                        idxb.at[pl.ds((1 - h) * _HSLOTS, _HSLOTS)])


        def _gather_channel(tab, buf):
            @pl.loop(0, _S)
            def _g_s(s):
                @plsc.parallel_loop(0, _P // 16, unroll=16)
                def _g(r):
                    n = plsc.load_gather(idxb, [iota33 + (r * 16 * _S + s)])
                    v = plsc.load_gather(
                        tab, [lax.shift_right_logical(n, 7),
                              jnp.bitwise_and(n, jnp.int32(127))])
                    buf[s, r // 8, pl.ds((r % 8) * 16, 16)] = v

        def _start_in(c, tab, sem):
            pltpu.async_copy(feat_hbm.at[b, c // 8, :, c % 8, :], tab, sem)

        def _wait_in(tab, sem):
            pltpu.make_async_copy(feat_hbm.at[0, 0, :, 0, :], tab, sem).wait()

        def _out_slice(c):
            return out_hbm.at[pl.ds((2 + c) * _S, _S), bt, :, bi, :]

        def _wait_out(buf, sem):
            pltpu.make_async_copy(buf, out_hbm.at[pl.ds(0, _S), 0, :, 0, :],
                                  sem).wait()

        def channels(cs, n):
            _start_in(cs, tab0, sem_in0)
            _start_in(cs + 1, tab1, sem_in1)
            _wait_in(tab0, sem_in0)
            _gather_channel(tab0, buf0)
            _start_in(cs + 2, tab0, sem_in0)
            pltpu.async_copy(buf0, _out_slice(cs), sem_out0)
            _wait_in(tab1, sem_in1)
            _gather_channel(tab1, buf1)
            _start_in(cs + 3, tab1, sem_in1)
            pltpu.async_copy(buf1, _out_slice(cs + 1), sem_out1)

            @pl.loop(2, n, step=2)
            def _pair(k):
                nxt0 = jnp.minimum(k + 2, n - 1) + cs
                nxt1 = jnp.minimum(k + 3, n - 1) + cs
                _wait_in(tab0, sem_in0)
                _wait_out(buf0, sem_out0)
                _gather_channel(tab0, buf0)
                _start_in(nxt0, tab0, sem_in0)
                pltpu.async_copy(buf0, _out_slice(cs + k), sem_out0)
                _wait_in(tab1, sem_in1)
                _wait_out(buf1, sem_out1)
                _gather_channel(tab1, buf1)
                _start_in(nxt1, tab1, sem_in1)
                pltpu.async_copy(buf1, _out_slice(cs + k + 1), sem_out1)

            _wait_in(tab0, sem_in0)
            _wait_in(tab1, sem_in1)
            _wait_out(buf0, sem_out0)
            _wait_out(buf1, sem_out1)

        channels(h * (_C // 2), _C // 2)

    return fused


def kernel(xyz, new_xyz, features, fps_idx):
    fused = _build_kernel()
    xyz_t = jnp.transpose(xyz, (0, 2, 1)).reshape(-1)
    cents = jnp.transpose(new_xyz, (0, 2, 1)).reshape(-1)
    feat = jnp.transpose(
        features.reshape(_B, _C // 8, 8, _N // 128, 128), (0, 1, 3, 2, 4))
    out6, _ = fused(xyz_t, cents, fps_idx.reshape(-1), feat)
    out = out6.reshape(_OUTC, _S, 2, 8, 8, 128)
    out = jnp.transpose(out, (2, 4, 0, 3, 5, 1))
    return out.reshape(_B, _OUTC, _P, _S)


# speedup vs baseline: 1.0064x; 1.0064x over previous
import functools

import jax
import jax.numpy as jnp
from jax import lax
from jax.experimental import pallas as pl
from jax.experimental.pallas import tpu as pltpu
from jax.experimental.pallas import tpu_sc as plsc

_B, _N, _P, _C = 16, 4096, 1024, 128
_S = 33
_OUTC = _C + 2
_R2 = 0.2 * 0.2
_NSLOTS = _P * _S
_HSLOTS = 512 * _S
_CH_SPLIT = 52


def _rsqrt(x, iters=3):
    i = lax.bitcast_convert_type(x, jnp.int32)
    i = jnp.int32(0x5F3759DF) - lax.shift_right_logical(i, 1)
    y = lax.bitcast_convert_type(i, jnp.float32)
    for _ in range(iters):
        y = y * (jnp.float32(1.5) - jnp.float32(0.5) * x * y * y)
    return y


def _acos(x):
    ax = jnp.abs(x)
    w = jnp.float32(1.0) - ax
    sw = w * _rsqrt(w)
    p = jnp.float32(-0.0012624911)
    for c in (0.0066700901, -0.0170881256, 0.0308918810, -0.0501743046,
              0.0889789874, -0.2145988016, 1.5707963050):
        p = p * ax + jnp.float32(c)
    t = sw * p
    return jnp.where(x >= jnp.float32(0.0), t, jnp.float32(3.1415927410125732) - t)


@functools.cache
def _build_kernel():
    mesh = plsc.VectorSubcoreMesh(core_axis_name="c", subcore_axis_name="s")

    @functools.partial(
        pl.kernel,
        out_type=(jax.ShapeDtypeStruct((_OUTC * _S, 2, 8, 8, 128), jnp.float32),
                  jax.ShapeDtypeStruct((_B * _P * _S,), jnp.int32)),
        mesh=mesh,
        compiler_params=pltpu.CompilerParams(needs_layout_passes=False),
        scratch_types=[
            pltpu.VMEM((3 * _N,), jnp.float32),
            pltpu.VMEM((3 * 512 + 16,), jnp.float32),
            pltpu.VMEM((512,), jnp.int32),
            pltpu.VMEM((_NSLOTS + 544,), jnp.int32),
            pltpu.VMEM((_S, 8, 128), jnp.float32),
            pltpu.VMEM((_S, 8, 128), jnp.float32),
            pltpu.VMEM((32, 128), jnp.float32),
            pltpu.VMEM((32, 128), jnp.float32),
            pltpu.SemaphoreType.DMA,
            pltpu.SemaphoreType.DMA,
            pltpu.SemaphoreType.DMA,
            pltpu.SemaphoreType.DMA,
        ],
    )
    def fused(xyz_hbm, cent_hbm, fps_hbm, feat_hbm, out_hbm, idx_hbm,
              xyzb, cbuf, fpsb, idxb, buf0, buf1, tab0, tab1,
              sem_in0, sem_in1, sem_out0, sem_out1):
        cid = lax.axis_index("c")
        sid = lax.axis_index("s")
        wid = cid * 16 + sid
        b = wid // 2
        h = wid % 2
        bt = b // 8
        bi = b % 8
        iota = lax.iota(jnp.int32, 16)
        iota33 = iota * _S
        ones16 = jnp.full((16,), 1, jnp.int32)
        r2 = jnp.float32(_R2)

        pltpu.sync_copy(xyz_hbm.at[pl.ds(b * (3 * _N), 3 * _N)], xyzb)
        for k in range(3):
            pltpu.sync_copy(
                cent_hbm.at[pl.ds(b * (3 * _P) + k * _P + h * 512, 512)],
                cbuf.at[pl.ds(k * 512, 512)])
        pltpu.sync_copy(fps_hbm.at[pl.ds(b * _P + h * 512, 512)], fpsb)

        def per_query(q, carry):
            cx = cbuf[pl.ds(q, 16)][0]
            cy = cbuf[pl.ds(512 + q, 16)][0]
            cz = cbuf[pl.ds(1024 + q, 16)][0]
            qv0 = h * _HSLOTS + q * _S
            lim = qv0 + 32

            def scan(lo, hi, cnt):
                @plsc.parallel_loop(lo, hi, unroll=8, carry=cnt)
                def supergroup(i, cnt):
                    base = i * 16
                    xv = xyzb[pl.ds(base, 16)]
                    yv = xyzb[pl.ds(_N + base, 16)]
                    zv = xyzb[pl.ds(2 * _N + base, 16)]
                    dx = xv - cx
                    dy = yv - cy
                    dz = zv - cz
                    d2 = dx * dx + dy * dy + dz * dz
                    m = d2 < r2
                    plsc.store_scatter(idxb,
                                       [cnt + plsc.cumsum(ones16, mask=m)],
                                       iota + base, mask=m)
                    return cnt + plsc.all_reduce_population_count(m)

                return supergroup

            def cond(st):
                g, cnt = st
                return jnp.logical_and(cnt[0] < lim, g < _N // 512)

            def body(st):
                g, cnt = st
                return g + jnp.int32(1), scan(g * 32, (g + 1) * 32, cnt)

            cnt = scan(0, 64, jnp.broadcast_to(qv0, (16,)))
            _, cnt = lax.while_loop(cond, body, (jnp.int32(2), cnt))
            cnt0 = cnt[0] - qv0
            first = jnp.where(cnt0 > 0,
                              idxb[pl.ds(qv0 + 1, 16)][0],
                              jnp.int32(0))
            firstv = jnp.broadcast_to(first, (16,))
            for u in range(2):
                lanes = iota + (u * 16)
                plsc.store_scatter(idxb, [qv0 + 1 + lanes], firstv,
                                   mask=lanes >= cnt0)
            return carry

        lax.fori_loop(0, 512, per_query, jnp.int32(0))

        @plsc.parallel_loop(0, 32)
        def _fps(i):
            fv = fpsb[pl.ds(i * 16, 16)]
            plsc.store_scatter(idxb, [h * _HSLOTS + (iota + i * 16) * _S], fv)

        @plsc.parallel_loop(0, 32)
        def _norm_centers(j):
            cx = cbuf[pl.ds(j * 16, 16)]
            cy = cbuf[pl.ds(512 + j * 16, 16)]
            cz = cbuf[pl.ds(1024 + j * 16, 16)]
            c2 = cx * cx + cy * cy + cz * cz
            n = c2 * _rsqrt(c2)
            r = _rsqrt(n + jnp.float32(1e-6))
            inv = r * r
            cbuf[pl.ds(j * 16, 16)] = cx * inv
            cbuf[pl.ds(512 + j * 16, 16)] = cy * inv
            cbuf[pl.ds(1024 + j * 16, 16)] = cz * inv

        @pl.loop(0, _S)
        def _inv_s(s):
            sbase = (h * 512) * _S + s

            @plsc.parallel_loop(0, 32, unroll=2)
            def _inv_chunk(r):
                pv = iota + r * 16
                n = plsc.load_gather(idxb, [iota33 + (r * 16 * _S + sbase)])
                gx = plsc.load_gather(xyzb, [n])
                gy = plsc.load_gather(xyzb, [n + _N])
                gz = plsc.load_gather(xyzb, [n + 2 * _N])
                cnx = plsc.load_gather(cbuf, [pv])
                cny = plsc.load_gather(cbuf, [pv + 512])
                cnz = plsc.load_gather(cbuf, [pv + 1024])
                s2 = gx * gx + gy * gy + gz * gz + jnp.float32(1e-12)
                rr = _rsqrt(s2)
                cross = (cnx * gx + cny * gy + cnz * gz) * rr
                cross = jnp.minimum(
                    jnp.maximum(cross, jnp.float32(-1 + 1e-6)),
                    jnp.float32(1 - 1e-6))
                pt = h * 4 + r // 8
                po = (r % 8) * 16
                buf0[s, pt, pl.ds(po, 16)] = s2 * rr
                buf1[s, pt, pl.ds(po, 16)] = _acos(cross)

        pltpu.sync_copy(buf0.at[:, pl.ds(h * 4, 4), :],
                        out_hbm.at[pl.ds(0, _S), bt, pl.ds(h * 4, 4), bi, :])
        pltpu.sync_copy(buf1.at[:, pl.ds(h * 4, 4), :],
                        out_hbm.at[pl.ds(_S, _S), bt, pl.ds(h * 4, 4), bi, :])

        base = (b * _P) * _S
        pltpu.sync_copy(idxb.at[pl.ds(h * _HSLOTS, _HSLOTS)],
                        idx_hbm.at[pl.ds(base + h * _HSLOTS, _HSLOTS)])
        plsc.subcore_barrier()
        pltpu.sync_copy(idx_hbm.at[pl.ds(base + (1 - h) * _HSLOTS, _HSLOTS)],
                        idxb.at[pl.ds((1 - h) * _HSLOTS, _HSLOTS)])


        def _gather_channel(tab, buf):
            @pl.loop(0, _S)
            def _g_s(s):
                @plsc.parallel_loop(0, _P // 16, unroll=16)
                def _g(r):
                    n = plsc.load_gather(idxb, [iota33 + (r * 16 * _S + s)])
                    v = plsc.load_gather(
                        tab, [lax.shift_right_logical(n, 7),
                              jnp.bitwise_and(n, jnp.int32(127))])
                    buf[s, r // 8, pl.ds((r % 8) * 16, 16)] = v

        def _start_in(c, tab, sem):
            pltpu.async_copy(feat_hbm.at[b, c // 8, :, c % 8, :], tab, sem)

        def _wait_in(tab, sem):
            pltpu.make_async_copy(feat_hbm.at[0, 0, :, 0, :], tab, sem).wait()

        def _out_slice(c):
            return out_hbm.at[pl.ds((2 + c) * _S, _S), bt, :, bi, :]

        def _wait_out(buf, sem):
            pltpu.make_async_copy(buf, out_hbm.at[pl.ds(0, _S), 0, :, 0, :],
                                  sem).wait()

        def channels(cs, n):
            _start_in(cs, tab0, sem_in0)
            _start_in(cs + 1, tab1, sem_in1)
            _wait_in(tab0, sem_in0)
            _gather_channel(tab0, buf0)
            _start_in(cs + 2, tab0, sem_in0)
            pltpu.async_copy(buf0, _out_slice(cs), sem_out0)
            _wait_in(tab1, sem_in1)
            _gather_channel(tab1, buf1)
            _start_in(cs + 3, tab1, sem_in1)
            pltpu.async_copy(buf1, _out_slice(cs + 1), sem_out1)

            @pl.loop(2, n, step=2)
            def _pair(k):
                nxt0 = jnp.minimum(k + 2, n - 1) + cs
                nxt1 = jnp.minimum(k + 3, n - 1) + cs
                _wait_in(tab0, sem_in0)
                _wait_out(buf0, sem_out0)
                _gather_channel(tab0, buf0)
                _start_in(nxt0, tab0, sem_in0)
                pltpu.async_copy(buf0, _out_slice(cs + k), sem_out0)
                _wait_in(tab1, sem_in1)
                _wait_out(buf1, sem_out1)
                _gather_channel(tab1, buf1)
                _start_in(nxt1, tab1, sem_in1)
                pltpu.async_copy(buf1, _out_slice(cs + k + 1), sem_out1)

            _wait_in(tab0, sem_in0)
            _wait_in(tab1, sem_in1)
            _wait_out(buf0, sem_out0)
            _wait_out(buf1, sem_out1)

        channels(h * (_C // 2), _C // 2)

    return fused


def kernel(xyz, new_xyz, features, fps_idx):
    fused = _build_kernel()
    xyz_t = jnp.transpose(xyz, (0, 2, 1)).reshape(-1)
    cents = jnp.transpose(new_xyz, (0, 2, 1)).reshape(-1)
    feat = jnp.transpose(
        features.reshape(_B, _C // 8, 8, _N // 128, 128), (0, 1, 3, 2, 4))
    out6, _ = fused(xyz_t, cents, fps_idx.reshape(-1), feat)
    out = out6.reshape(_OUTC, _S, 2, 8, 8, 128)
    out = jnp.transpose(out, (2, 4, 0, 3, 5, 1))
    return out.reshape(_B, _OUTC, _P, _S)
